# baseline (speedup 1.0000x reference)
import jax
import jax.numpy as jnp
from jax.experimental import pallas as pl
from jax.experimental.pallas import tpu as pltpu


N_TOK = 8192
D_MODEL = 4096
N_GATES = 64
R = 256
NCHUNK = N_TOK // R
NBUF = 8


def _router_kernel(x_hbm, wt_ref, probs_ref, scores_ref, idx_ref, bufs, sems):
    def start_copy(c):
        slot = c % NBUF
        pltpu.make_async_copy(
            x_hbm.at[pl.ds(c * R, R), :], bufs.at[slot], sems.at[slot]
        ).start()

    def wait_copy(c):
        slot = c % NBUF
        pltpu.make_async_copy(
            x_hbm.at[pl.ds(c * R, R), :], bufs.at[slot], sems.at[slot]
        ).wait()

    for c in range(min(NBUF, NCHUNK)):
        start_copy(c)

    wt = wt_ref[...]
    for c in range(NCHUNK):
        slot = c % NBUF
        wait_copy(c)
        logits = jnp.dot(bufs[slot], wt, preferred_element_type=jnp.float32)
        m = jnp.max(logits, axis=-1, keepdims=True)
        e = jnp.exp(logits - m)
        s = jnp.sum(e, axis=-1, keepdims=True)
        probs = e / s
        probs_ref[pl.ds(c * R, R), :] = probs
        scores_ref[c, :] = jnp.max(probs, axis=-1)
        idx_ref[c, :] = jnp.argmax(probs, axis=-1).astype(jnp.int32)
        if c + NBUF < NCHUNK:
            start_copy(c + NBUF)


@jax.jit
def kernel(x, W):
    wt = W.T
    probs, scores, idx = pl.pallas_call(
        _router_kernel,
        grid=(),
        in_specs=[
            pl.BlockSpec(memory_space=pltpu.MemorySpace.HBM),
            pl.BlockSpec(memory_space=pltpu.MemorySpace.VMEM),
        ],
        out_specs=[
            pl.BlockSpec(memory_space=pltpu.MemorySpace.VMEM),
            pl.BlockSpec(memory_space=pltpu.MemorySpace.VMEM),
            pl.BlockSpec(memory_space=pltpu.MemorySpace.VMEM),
        ],
        out_shape=[
            jax.ShapeDtypeStruct((N_TOK, N_GATES), jnp.float32),
            jax.ShapeDtypeStruct((NCHUNK, R), jnp.float32),
            jax.ShapeDtypeStruct((NCHUNK, R), jnp.int32),
        ],
        scratch_shapes=[
            pltpu.VMEM((NBUF, R, D_MODEL), jnp.float32),
            pltpu.SemaphoreType.DMA((NBUF,)),
        ],
    )(x, wt)
    return idx.reshape(N_TOK), scores.reshape(N_TOK), probs


# speedup vs baseline: 1.1423x; 1.1423x over previous
import jax
import jax.numpy as jnp
from jax.experimental import pallas as pl
from jax.experimental.pallas import tpu as pltpu


N_TOK = 8192
D_MODEL = 4096
N_GATES = 64
R = 512
NCHUNK = N_TOK // R
NQ = 4


def _router_kernel(x_hbm, wt_ref, probs_ref, scores_ref, idx_ref, *scratch):
    bufs = scratch[:NQ]
    sems = scratch[NQ:]

    def start_copy(c):
        q = c % NQ
        pltpu.make_async_copy(
            x_hbm.at[pl.ds(c * R, R), :], bufs[q], sems[q]
        ).start()

    def wait_copy(c):
        q = c % NQ
        pltpu.make_async_copy(
            x_hbm.at[pl.ds(c * R, R), :], bufs[q], sems[q]
        ).wait()

    for c in range(min(NQ, NCHUNK)):
        start_copy(c)

    wt = wt_ref[...]
    for c in range(NCHUNK):
        q = c % NQ
        wait_copy(c)
        logits = jnp.dot(bufs[q][...], wt, preferred_element_type=jnp.float32)
        m = jnp.max(logits, axis=-1, keepdims=True)
        e = jnp.exp(logits - m)
        s = jnp.sum(e, axis=-1, keepdims=True)
        probs = e / s
        probs_ref[pl.ds(c * R, R), :] = probs
        scores_ref[c, :] = jnp.max(probs, axis=-1)
        idx_ref[c, :] = jnp.argmax(probs, axis=-1).astype(jnp.int32)
        if c + NQ < NCHUNK:
            start_copy(c + NQ)


@jax.jit
def kernel(x, W):
    wt = W.T
    probs, scores, idx = pl.pallas_call(
        _router_kernel,
        grid=(),
        in_specs=[
            pl.BlockSpec(memory_space=pltpu.MemorySpace.HBM),
            pl.BlockSpec(memory_space=pltpu.MemorySpace.VMEM),
        ],
        out_specs=[
            pl.BlockSpec(memory_space=pltpu.MemorySpace.VMEM),
            pl.BlockSpec(memory_space=pltpu.MemorySpace.VMEM),
            pl.BlockSpec(memory_space=pltpu.MemorySpace.VMEM),
        ],
        out_shape=[
            jax.ShapeDtypeStruct((N_TOK, N_GATES), jnp.float32),
            jax.ShapeDtypeStruct((NCHUNK, R), jnp.float32),
            jax.ShapeDtypeStruct((NCHUNK, R), jnp.int32),
        ],
        scratch_shapes=[pltpu.VMEM((R, D_MODEL), jnp.float32) for _ in range(NQ)]
        + [pltpu.SemaphoreType.DMA for _ in range(NQ)],
    )(x, wt)
    return idx.reshape(N_TOK), scores.reshape(N_TOK), probs
